# baseline (speedup 1.0000x reference)
import functools

import jax
import jax.numpy as jnp
from jax import lax
from jax.experimental import pallas as pl
from jax.experimental.pallas import tpu as pltpu
from jax.experimental.pallas import tpu_sc as plsc

_NC = 2
_NS = 16
_NW = _NC * _NS

_IPG = 128
_K = 8
_C = _K * _IPG


def _body(table_hbm, idx_hbm, out_hbm, idx_v, rows_v, sem):
    wid = lax.axis_index("s") * _NC + lax.axis_index("c")
    nblk = idx_hbm.shape[0] // _NW
    blk0 = wid * nblk

    @pl.loop(0, nblk // _K)
    def _group(g):
        row0 = blk0 + g * _K
        pltpu.sync_copy(idx_hbm.at[pl.ds(row0, _K)], idx_v)
        copies = [
            pltpu.async_copy(
                table_hbm.at[idx_v.at[j]],
                rows_v.at[pl.ds(j * _IPG, _IPG)],
                sem,
            )
            for j in range(_K)
        ]
        for cp in copies:
            cp.wait()
        pltpu.sync_copy(rows_v, out_hbm.at[pl.ds(row0 * _IPG, _C)])


@functools.partial(jax.jit, static_argnames=("interpret",))
def _gather(averages, idx2d, interpret=False):
    b = idx2d.shape[0] * _IPG
    d = averages.shape[1]
    mesh = plsc.VectorSubcoreMesh(core_axis_name="c", subcore_axis_name="s")
    return pl.kernel(
        _body,
        out_type=jax.ShapeDtypeStruct((b, d), averages.dtype),
        mesh=mesh,
        scratch_types=[
            pltpu.VMEM((_K, _IPG), jnp.int32),
            pltpu.VMEM((_C, 64), jnp.float32),
            pltpu.SemaphoreType.DMA,
        ],
        compiler_params=pltpu.CompilerParams(use_tc_tiling_on_sc=False),
        interpret=interpret,
    )(averages, idx2d)


def kernel(indicies, averages):
    batch, seq = indicies.shape
    d = averages.shape[1]
    idx2d = indicies.reshape(-1, _IPG).astype(jnp.int32)
    out = _gather(averages, idx2d)
    return out.reshape(batch, seq, d)


# speedup vs baseline: 1.0309x; 1.0309x over previous
import functools

import jax
import jax.numpy as jnp
from jax import lax
from jax.experimental import pallas as pl
from jax.experimental.pallas import tpu as pltpu
from jax.experimental.pallas import tpu_sc as plsc

_NC = 2
_NS = 16
_NW = _NC * _NS

_IPG = 128
_K = 5
_C = _K * _IPG


def _body(table_hbm, idx_hbm, out_hbm, idx_v, rows_v, gsem, osem0, osem1):
    wid = lax.axis_index("s") * _NC + lax.axis_index("c")
    nblk = idx_hbm.shape[0] // _NW
    blk0 = wid * nblk
    ng = nblk // _K
    osems = (osem0, osem1)

    pltpu.sync_copy(idx_hbm.at[pl.ds(blk0, _K)], idx_v.at[0])

    @pl.loop(0, ng, step=2)
    def _pair(p):
        for b in range(2):
            g = p + b
            row0 = blk0 + g * _K
            rows = rows_v.at[b]
            out_slice = out_hbm.at[pl.ds(row0 * _IPG, _C)]

            @pl.when(g >= 2)
            def _():
                pltpu.make_async_copy(rows, out_slice, osems[b]).wait()

            copies = [
                pltpu.async_copy(
                    table_hbm.at[idx_v.at[b, j]],
                    rows.at[pl.ds(j * _IPG, _IPG)],
                    gsem,
                )
                for j in range(_K)
            ]

            @pl.when(g + 1 < ng)
            def _():
                pltpu.sync_copy(
                    idx_hbm.at[pl.ds(row0 + _K, _K)], idx_v.at[1 - b]
                )

            for cp in copies:
                cp.wait()

            pltpu.async_copy(rows, out_slice, osems[b])

    for b in range(2):
        pltpu.make_async_copy(
            rows_v.at[b], out_hbm.at[pl.ds(blk0 * _IPG, _C)], osems[b]
        ).wait()


@functools.partial(jax.jit, static_argnames=("interpret",))
def _gather(averages, idx2d, interpret=False):
    b = idx2d.shape[0] * _IPG
    d = averages.shape[1]
    mesh = plsc.VectorSubcoreMesh(core_axis_name="c", subcore_axis_name="s")
    return pl.kernel(
        _body,
        out_type=jax.ShapeDtypeStruct((b, d), averages.dtype),
        mesh=mesh,
        scratch_types=[
            pltpu.VMEM((2, _K, _IPG), jnp.int32),
            pltpu.VMEM((2, _C, 64), jnp.float32),
            pltpu.SemaphoreType.DMA,
            pltpu.SemaphoreType.DMA,
            pltpu.SemaphoreType.DMA,
        ],
        compiler_params=pltpu.CompilerParams(use_tc_tiling_on_sc=False),
        interpret=interpret,
    )(averages, idx2d)


def kernel(indicies, averages):
    batch, seq = indicies.shape
    d = averages.shape[1]
    idx2d = indicies.reshape(-1, _IPG).astype(jnp.int32)
    out = _gather(averages, idx2d)
    return out.reshape(batch, seq, d)
